# baseline (speedup 1.0000x reference)
import functools

import jax
import jax.numpy as jnp
from jax import lax
from jax.experimental import pallas as pl
from jax.experimental.pallas import tpu as pltpu
from jax.experimental.pallas import tpu_sc as plsc

_NUM_CLASSES = 100000
_FEAT_DIM = 64
_BATCH = 16384

_INFO = plsc.get_sparse_core_info()
_NC = _INFO.num_cores
_NS = _INFO.num_subcores
_LANES = _INFO.num_lanes
_NW = _NC * _NS
_BPW = _BATCH // _NW
_CHUNK = 128
_NCHUNK = _BPW // _CHUNK


def _sc_body(features_hbm, labels_hbm, centers_hbm, out_hbm,
             idx_v, rows_v, feats_v, acc_v, gsem, fsem):
    wid = lax.axis_index("s") * _NC + lax.axis_index("c")
    base = wid * _BPW

    lcopies = [
        pltpu.async_copy(labels_hbm.at[pl.ds(base + j * _CHUNK, _CHUNK)],
                         idx_v.at[j], gsem.at[j])
        for j in range(_NCHUNK)
    ]
    fcopy = pltpu.async_copy(
        features_hbm.at[pl.ds(base * _FEAT_DIM, _BPW * _FEAT_DIM)],
        feats_v, fsem)

    gathers = []
    for j in range(_NCHUNK):
        lcopies[j].wait()
        gathers.append(
            pltpu.async_copy(centers_hbm.at[idx_v.at[j]],
                             rows_v.at[pl.ds(j * _CHUNK, _CHUNK)],
                             gsem.at[j]))
    fcopy.wait()

    zero = jnp.zeros((_LANES,), jnp.float32)
    nacc = _FEAT_DIM // _LANES

    def body(i, accs):
        out = []
        for k in range(nacc):
            f = feats_v[pl.ds(i * _FEAT_DIM + k * _LANES, _LANES)]
            c = rows_v[i, pl.ds(k * _LANES, _LANES)]
            d = f - c
            out.append(accs[k] + d * d)
        return tuple(out)

    accs = (zero,) * nacc
    for j in range(_NCHUNK):
        gathers[j].wait()
        accs = lax.fori_loop(j * _CHUNK, (j + 1) * _CHUNK, body, accs)
    total = (accs[0] + accs[1]) + (accs[2] + accs[3])
    acc_v[...] = total * jnp.float32(1.0 / (_BATCH * _FEAT_DIM))
    pltpu.sync_copy(acc_v, out_hbm.at[wid])


@functools.partial(jax.jit, static_argnames=())
def kernel(features, labels, centers):
    mesh = plsc.VectorSubcoreMesh(core_axis_name="c", subcore_axis_name="s")
    partials = pl.kernel(
        _sc_body,
        mesh=mesh,
        out_type=jax.ShapeDtypeStruct((_NW, _LANES), jnp.float32),
        scratch_types=[
            pltpu.VMEM((_NCHUNK, _CHUNK), jnp.int32),
            pltpu.VMEM((_BPW, _FEAT_DIM), jnp.float32),
            pltpu.VMEM((_BPW * _FEAT_DIM,), jnp.float32),
            pltpu.VMEM((_LANES,), jnp.float32),
            pltpu.SemaphoreType.DMA((_NCHUNK,)),
            pltpu.SemaphoreType.DMA,
        ],
        compiler_params=pltpu.CompilerParams(use_tc_tiling_on_sc=False),
    )(features.reshape(_BATCH * _FEAT_DIM), labels.astype(jnp.int32),
      centers)
    return jnp.sum(partials)
